# baseline (speedup 1.0000x reference)
import functools

import jax
import jax.numpy as jnp
from jax import lax
from jax.experimental import pallas as pl
from jax.experimental.pallas import tpu as pltpu
from jax.experimental.pallas import tpu_sc as plsc

B = 4
S = 2048
D = 768

_info = plsc.get_sparse_core_info()
_NC = _info.num_cores
_NS = _info.num_subcores
_L = _info.num_lanes
_NW = _NC * _NS
_S_PER_W = S // _NW
_C = 16
_CPB = _S_PER_W // _C
_NCHUNK = B * _CPB
_NBUF = 6
_VECS = D // _L

_mesh = plsc.VectorSubcoreMesh(core_axis_name="c", subcore_axis_name="s")


@functools.partial(
    pl.kernel,
    mesh=_mesh,
    out_type=jax.ShapeDtypeStruct((B, S, D), jnp.float32),
    scratch_types=[
        pltpu.VMEM((B, _S_PER_W), jnp.int32),
        pltpu.VMEM((_S_PER_W, D), jnp.float32),
    ] + [pltpu.VMEM((_C, D), jnp.float32) for _ in range(_NBUF)] + [
        pltpu.SemaphoreType.DMA((_NBUF,)),
        pltpu.SemaphoreType.DMA((_NBUF,)),
        pltpu.SemaphoreType.DMA,
    ],
)
def _embed(ids_hbm, word_hbm, pos_hbm, out_hbm,
           idx_v, pos_v, w0, w1, w2, w3, w4, w5, gsem, wsem, psem):
    wid = lax.axis_index("s") * _NC + lax.axis_index("c")
    s_base = wid * _S_PER_W
    w = (w0, w1, w2, w3, w4, w5)

    def chunk_bh(j):
        return j // _CPB, (j % _CPB) * _C

    pos_load = pltpu.async_copy(pos_hbm.at[pl.ds(s_base, _S_PER_W)],
                                pos_v, psem)
    for b in range(B):
        pltpu.sync_copy(ids_hbm.at[b, pl.ds(s_base, _S_PER_W)], idx_v.at[b])

    def fire_gather(j):
        b, h = chunk_bh(j)
        return pltpu.async_copy(
            word_hbm.at[idx_v.at[b, pl.ds(h, _C)]], w[j % _NBUF],
            gsem.at[j % _NBUF])

    gathers = [None] * _NCHUNK
    writes = [None] * _NCHUNK
    for j in range(_NBUF - 2):
        gathers[j] = fire_gather(j)

    pos_load.wait()

    for j in range(_NCHUNK):
        cur = j % _NBUF
        gathers[j].wait()
        b, h = chunk_bh(j)

        @plsc.parallel_loop(0, _C)
        def _row(r, cur=cur, h=h):
            @plsc.parallel_loop(0, _VECS)
            def _vec(c):
                sl = pl.ds(c * _L, _L)
                plsc.addupdate(w[cur].at[r, sl], pos_v[h + r, sl])
        writes[j] = pltpu.async_copy(
            w[cur], out_hbm.at[b, pl.ds(s_base + h, _C)], wsem.at[cur])

        nxt = j + _NBUF - 2
        if nxt < _NCHUNK:
            if j >= 2:
                writes[j - 2].wait()
            gathers[nxt] = fire_gather(nxt)

    for j in range(_NCHUNK - _NBUF, _NCHUNK):
        writes[j].wait()


def kernel(ids, word_table, pos_table):
    return _embed(ids.astype(jnp.int32), word_table, pos_table)


# speedup vs baseline: 1.2730x; 1.2730x over previous
import functools

import jax
import jax.numpy as jnp
from jax import lax
from jax.experimental import pallas as pl
from jax.experimental.pallas import tpu as pltpu
from jax.experimental.pallas import tpu_sc as plsc

B = 4
S = 2048
D = 768

_info = plsc.get_sparse_core_info()
_NC = _info.num_cores
_NS = _info.num_subcores
_L = _info.num_lanes
_NW = _NC * _NS
_S_PER_W = S // _NW
_C = 32
_CPB = _S_PER_W // _C
_NCHUNK = B * _CPB
_NBUF = 3
_VECS = D // _L

_mesh = plsc.VectorSubcoreMesh(core_axis_name="c", subcore_axis_name="s")


@functools.partial(
    pl.kernel,
    mesh=_mesh,
    out_type=jax.ShapeDtypeStruct((B, S, D), jnp.float32),
    scratch_types=[
        pltpu.VMEM((B, _S_PER_W), jnp.int32),
        pltpu.VMEM((_S_PER_W, D), jnp.float32),
    ] + [pltpu.VMEM((_C, D), jnp.float32) for _ in range(_NBUF)] + [
        pltpu.SemaphoreType.DMA((_NBUF,)),
        pltpu.SemaphoreType.DMA((_NBUF,)),
        pltpu.SemaphoreType.DMA,
    ],
)
def _embed(ids_hbm, word_hbm, pos_hbm, out_hbm,
           idx_v, pos_v, w0, w1, w2, gsem, wsem, psem):
    wid = lax.axis_index("s") * _NC + lax.axis_index("c")
    s_base = wid * _S_PER_W
    w = (w0, w1, w2)

    def chunk_bh(j):
        return j // _CPB, (j % _CPB) * _C

    pos_load = pltpu.async_copy(pos_hbm.at[pl.ds(s_base, _S_PER_W)],
                                pos_v, psem)
    for b in range(B):
        pltpu.sync_copy(ids_hbm.at[b, pl.ds(s_base, _S_PER_W)], idx_v.at[b])

    def fire_gather(j):
        b, h = chunk_bh(j)
        return pltpu.async_copy(
            word_hbm.at[idx_v.at[b, pl.ds(h, _C)]], w[j % _NBUF],
            gsem.at[j % _NBUF])

    gathers = [None] * _NCHUNK
    writes = [None] * _NCHUNK
    for j in range(_NBUF - 1):
        gathers[j] = fire_gather(j)

    pos_load.wait()

    for j in range(_NCHUNK):
        cur = j % _NBUF
        gathers[j].wait()
        b, h = chunk_bh(j)

        def _row(r, carry, cur=cur, h=h):
            for c in range(_VECS):
                sl = pl.ds(c * _L, _L)
                plsc.addupdate(w[cur].at[r, sl], pos_v[h + r, sl])
            return carry

        lax.fori_loop(0, _C, _row, 0)
        writes[j] = pltpu.async_copy(
            w[cur], out_hbm.at[b, pl.ds(s_base + h, _C)], wsem.at[cur])

        nxt = j + _NBUF - 1
        if nxt < _NCHUNK:
            if j >= 1:
                writes[j - 1].wait()
            gathers[nxt] = fire_gather(nxt)

    for j in range(_NCHUNK - _NBUF, _NCHUNK):
        writes[j].wait()


def kernel(ids, word_table, pos_table):
    return _embed(ids.astype(jnp.int32), word_table, pos_table)
